# Initial kernel scaffold; baseline (speedup 1.0000x reference)
import jax
import jax.numpy as jnp
from jax.experimental import pallas as pl


def kernel(user_emb, item_emb, group_emb, W_agg, b_agg, uh_row, uh_col, uh_val, ih_row, ih_col, ih_val, fh_row, fh_col, fh_val):
    raise NotImplementedError("write your pallas kernel here")


# speedup vs baseline: 1.0093x; 1.0093x over previous
import functools

import jax
import jax.numpy as jnp
from jax import lax
from jax.experimental import pallas as pl
from jax.experimental.pallas import tpu as pltpu

NUM_USERS = 50000
NUM_ITEMS = 50000
NUM_GROUPS = 10000
NUM_UI = NUM_USERS + NUM_ITEMS
EMB = 128


def _agg_matmul_body(um_ref, im_ref, w_ref, b_ref, g_ref, msg_ref, gout_ref):
    x = jnp.concatenate([um_ref[...], im_ref[...]], axis=1)
    msg = (
        jax.lax.dot_general(
            x, w_ref[...], (((1,), (0,)), ((), ())),
            preferred_element_type=jnp.float32,
            precision=jax.lax.Precision.HIGHEST,
        )
        + b_ref[...]
    )
    msg_ref[...] = msg
    gout_ref[...] = g_ref[...] + msg


def _agg_matmul(user_msg, item_msg, w, b, g_acc):
    blk = 2000
    grid = (NUM_GROUPS // blk,)
    return pl.pallas_call(
        _agg_matmul_body,
        grid=grid,
        in_specs=[
            pl.BlockSpec((blk, EMB), lambda i: (i, 0)),
            pl.BlockSpec((blk, EMB), lambda i: (i, 0)),
            pl.BlockSpec((2 * EMB, EMB), lambda i: (0, 0)),
            pl.BlockSpec((1, EMB), lambda i: (0, 0)),
            pl.BlockSpec((blk, EMB), lambda i: (i, 0)),
        ],
        out_specs=[
            pl.BlockSpec((blk, EMB), lambda i: (i, 0)),
            pl.BlockSpec((blk, EMB), lambda i: (i, 0)),
        ],
        out_shape=[
            jax.ShapeDtypeStruct((NUM_GROUPS, EMB), jnp.float32),
            jax.ShapeDtypeStruct((NUM_GROUPS, EMB), jnp.float32),
        ],
    )(user_msg, item_msg, w, b, g_acc)


def _ui_sum_body(u_ref, i_ref, e1_ref, e2_ref, o_ref):
    half = pl.program_id(1)
    base = jnp.where(half == 0, u_ref[...], i_ref[...])
    o_ref[...] = base + e1_ref[...] + e2_ref[...]


def _ui_sum(user_emb, item_emb, emb1, emb2):
    blk = 2000
    nb = NUM_USERS // blk
    return pl.pallas_call(
        _ui_sum_body,
        grid=(nb, 2),
        in_specs=[
            pl.BlockSpec((blk, EMB), lambda i, h: (i, 0)),
            pl.BlockSpec((blk, EMB), lambda i, h: (i, 0)),
            pl.BlockSpec((blk, EMB), lambda i, h: (i + h * nb, 0)),
            pl.BlockSpec((blk, EMB), lambda i, h: (i + h * nb, 0)),
        ],
        out_specs=pl.BlockSpec((blk, EMB), lambda i, h: (i + h * nb, 0)),
        out_shape=jax.ShapeDtypeStruct((NUM_UI, EMB), jnp.float32),
    )(user_emb, item_emb, emb1, emb2)


def _spmm(row, col, vals, dense, n_rows):
    gathered = vals[:, None] * jnp.take(dense, col, axis=0)
    return jax.ops.segment_sum(gathered, row, num_segments=n_rows)


def kernel(user_emb, item_emb, group_emb, W_agg, b_agg,
           uh_row, uh_col, uh_val, ih_row, ih_col, ih_val,
           fh_row, fh_col, fh_val):
    ue, ie = user_emb, item_emb
    g_acc = group_emb
    embs = []
    for i in range(2):
        user_msg = _spmm(uh_row, uh_col, uh_val, ue, NUM_GROUPS)
        item_msg = _spmm(ih_row, ih_col, ih_val, ie, NUM_GROUPS)
        msg, g_acc = _agg_matmul(user_msg, item_msg, W_agg[i],
                                 b_agg[i].reshape(1, EMB), g_acc)
        emb = _spmm(fh_row, fh_col, fh_val, msg, NUM_UI)
        ue = emb[:NUM_USERS]
        ie = emb[NUM_USERS:]
        embs.append(emb)
    final_ui = _ui_sum(user_emb, item_emb, embs[0], embs[1])
    return (final_ui, g_acc)


# speedup vs baseline: 1.3707x; 1.3580x over previous
import functools

import jax
import jax.numpy as jnp
from jax import lax
from jax.experimental import pallas as pl
from jax.experimental.pallas import tpu as pltpu
from jax.experimental.pallas import tpu_sc as plsc

NUM_USERS = 50000
NUM_ITEMS = 50000
NUM_GROUPS = 10000
NUM_UI = NUM_USERS + NUM_ITEMS
EMB = 128
UH_NNZ = 320000

_A_PER_TILE = UH_NNZ // 16
_A_CH = 80
_A_NCH = _A_PER_TILE // _A_CH


_GDN = lax.GatherDimensionNumbers(
    offset_dims=(), collapsed_slice_dims=(0,), start_index_map=(0,))


def _bcast_lane(vec, j):
    idx = jnp.full((16, 1), j, jnp.int32)
    return lax.gather(vec, idx, _GDN, (1,),
                      mode=lax.GatherScatterMode.PROMISE_IN_BOUNDS)


def _scale_chunk(rows_v, val_v, ch):
    for g in range(ch // 16):
        vals = val_v[pl.ds(g * 16, 16)]
        for j in range(16):
            vj = _bcast_lane(vals, j)
            jj = g * 16 + j
            for k in range(EMB // 16):
                sl = pl.ds(k * 16, 16)
                rows_v[jj, sl] = rows_v[jj, sl] * vj


def _group_spmm_body(table, cols, vals, rows, zeros, out,
                     acc, col_v, val_v, row_v, rows_v):
    c = lax.axis_index("c")
    s = lax.axis_index("s")
    @pl.when(s < 15)
    def _():
        pltpu.sync_copy(zeros.at[pl.ds(0, 624)], acc.at[pl.ds(s * 624, 624)])

    @pl.when(s == 15)
    def _():
        pltpu.sync_copy(zeros.at[pl.ds(0, 640)], acc.at[pl.ds(9360, 640)])

    plsc.subcore_barrier()

    base = c * UH_NNZ + s * _A_PER_TILE

    def chunk(i, _):
        lo = base + i * _A_CH
        pltpu.sync_copy(cols.at[pl.ds(lo, _A_CH)], col_v)
        pltpu.sync_copy(vals.at[pl.ds(lo, _A_CH)], val_v)
        pltpu.sync_copy(rows.at[pl.ds(lo, _A_CH)], row_v)
        pltpu.sync_copy(table.at[col_v], rows_v)
        _scale_chunk(rows_v, val_v, _A_CH)
        pltpu.sync_copy(rows_v, acc.at[row_v], add=True)
        return 0

    lax.fori_loop(0, _A_NCH, chunk, 0)
    plsc.subcore_barrier()

    @pl.when(s < 15)
    def _():
        pltpu.sync_copy(acc.at[pl.ds(s * 624, 624)],
                        out.at[c, pl.ds(s * 624, 624)])

    @pl.when(s == 15)
    def _():
        pltpu.sync_copy(acc.at[pl.ds(9360, 640)],
                        out.at[c, pl.ds(9360, 640)])


def _group_spmm(table, cols, vals, rows, zeros):
    mesh = plsc.VectorSubcoreMesh(core_axis_name="c", subcore_axis_name="s")
    return pl.kernel(
        _group_spmm_body,
        mesh=mesh,
        out_type=jax.ShapeDtypeStruct((2, NUM_GROUPS, EMB), jnp.float32),
        scratch_types=[
            pltpu.VMEM_SHARED((NUM_GROUPS, EMB), jnp.float32),
            pltpu.VMEM((_A_CH,), jnp.int32),
            pltpu.VMEM((_A_CH,), jnp.float32),
            pltpu.VMEM((_A_CH,), jnp.int32),
            pltpu.VMEM((_A_CH, EMB), jnp.float32),
        ],
    )(table, cols, vals, rows, zeros)


def _agg_matmul_body(um_ref, im_ref, w_ref, b_ref, g_ref, msg_ref, gout_ref):
    x = jnp.concatenate([um_ref[...], im_ref[...]], axis=1)
    msg = (
        jax.lax.dot_general(
            x, w_ref[...], (((1,), (0,)), ((), ())),
            preferred_element_type=jnp.float32,
            precision=jax.lax.Precision.HIGHEST,
        )
        + b_ref[...]
    )
    msg_ref[...] = msg
    gout_ref[...] = g_ref[...] + msg


def _agg_matmul(user_msg, item_msg, w, b, g_acc):
    blk = 2000
    grid = (NUM_GROUPS // blk,)
    return pl.pallas_call(
        _agg_matmul_body,
        grid=grid,
        in_specs=[
            pl.BlockSpec((blk, EMB), lambda i: (i, 0)),
            pl.BlockSpec((blk, EMB), lambda i: (i, 0)),
            pl.BlockSpec((2 * EMB, EMB), lambda i: (0, 0)),
            pl.BlockSpec((1, EMB), lambda i: (0, 0)),
            pl.BlockSpec((blk, EMB), lambda i: (i, 0)),
        ],
        out_specs=[
            pl.BlockSpec((blk, EMB), lambda i: (i, 0)),
            pl.BlockSpec((blk, EMB), lambda i: (i, 0)),
        ],
        out_shape=[
            jax.ShapeDtypeStruct((NUM_GROUPS, EMB), jnp.float32),
            jax.ShapeDtypeStruct((NUM_GROUPS, EMB), jnp.float32),
        ],
    )(user_msg, item_msg, w, b, g_acc)


def _ui_sum_body(u_ref, i_ref, e1_ref, e2_ref, o_ref):
    half = pl.program_id(1)
    base = jnp.where(half == 0, u_ref[...], i_ref[...])
    o_ref[...] = base + e1_ref[...] + e2_ref[...]


def _ui_sum(user_emb, item_emb, emb1, emb2):
    blk = 2000
    nb = NUM_USERS // blk
    return pl.pallas_call(
        _ui_sum_body,
        grid=(nb, 2),
        in_specs=[
            pl.BlockSpec((blk, EMB), lambda i, h: (i, 0)),
            pl.BlockSpec((blk, EMB), lambda i, h: (i, 0)),
            pl.BlockSpec((blk, EMB), lambda i, h: (i + h * nb, 0)),
            pl.BlockSpec((blk, EMB), lambda i, h: (i + h * nb, 0)),
        ],
        out_specs=pl.BlockSpec((blk, EMB), lambda i, h: (i + h * nb, 0)),
        out_shape=jax.ShapeDtypeStruct((NUM_UI, EMB), jnp.float32),
    )(user_emb, item_emb, emb1, emb2)


def _spmm(row, col, vals, dense, n_rows):
    gathered = vals[:, None] * jnp.take(dense, col, axis=0)
    return jax.ops.segment_sum(gathered, row, num_segments=n_rows)


def kernel(user_emb, item_emb, group_emb, W_agg, b_agg,
           uh_row, uh_col, uh_val, ih_row, ih_col, ih_val,
           fh_row, fh_col, fh_val):
    i32 = jnp.int32
    ui0 = jnp.concatenate([user_emb, item_emb], axis=0)
    a_cols = jnp.concatenate([uh_col.astype(i32),
                              ih_col.astype(i32) + NUM_USERS])
    a_vals = jnp.concatenate([uh_val, ih_val])
    a_rows = jnp.concatenate([uh_row.astype(i32), ih_row.astype(i32)])
    zeros = jnp.zeros((12500, EMB), jnp.float32)

    table = ui0
    g_acc = group_emb
    embs = []
    for i in range(2):
        msgs = _group_spmm(table, a_cols, a_vals, a_rows, zeros)
        msg, g_acc = _agg_matmul(msgs[0], msgs[1], W_agg[i],
                                 b_agg[i].reshape(1, EMB), g_acc)
        emb = _spmm(fh_row, fh_col, fh_val, msg, NUM_UI)
        table = emb
        embs.append(emb)
    final_ui = _ui_sum(user_emb, item_emb, embs[0], embs[1])
    return (final_ui, g_acc)
